# baseline (speedup 1.0000x reference)
import jax
import jax.numpy as jnp
from jax import lax
from jax.experimental import pallas as pl
from jax.experimental.pallas import tpu as pltpu
from jax.experimental.pallas import tpu_sc as plsc

_B, _L, _D = 16384, 200, 10
_DP = 16
_NC, _NS = 2, 16
_NW = _NC * _NS
_BSPAN = _B // _NW
_LT = 8
_NSTRIP = _L // _LT
_NJ = _LT * _BSPAN // 16


def _body(yt_hbm, emb_hbm, w_hbm, b_hbm, q_hbm,
          ytv0, ytv1, qv0, qv1, embv, wv, bv, lut, si0, si1, so0, so1):
    wid = lax.axis_index("s") * _NC + lax.axis_index("c")
    bt0 = wid * (_BSPAN // 128)
    ytv, qv, si, so = [ytv0, ytv1], [qv0, qv1], [si0, si1], [so0, so1]

    in_h = [None, None]
    in_h[0] = pltpu.async_copy(
        yt_hbm.at[0, pl.ds(bt0, _BSPAN // 128)], ytv[0], si[0])

    pltpu.sync_copy(emb_hbm, embv)
    pltpu.sync_copy(w_hbm, wv)
    pltpu.sync_copy(b_hbm, bv)
    for i in range(10):
        acc = bv[...]
        for k in range(10):
            e = embv[pl.ds((i * 10 + k) * _DP, _DP)]
            acc = acc + e * wv[pl.ds(k * _DP, _DP)]
        lut[pl.ds(i * _DP, _DP)] = jnp.maximum(acc, 0.0)

    out_h = [[], []]
    for c in range(_NSTRIP):
        p = c % 2
        if c + 1 < _NSTRIP:
            in_h[1 - p] = pltpu.async_copy(
                yt_hbm.at[c + 1, pl.ds(bt0, _BSPAN // 128)],
                ytv[1 - p], si[1 - p])
        in_h[p].wait()
        for h in out_h[p]:
            h.wait()
        out_h[p] = []

        ytv_p, qv_p = ytv[p], qv[p]

        @plsc.parallel_loop(0, _NJ, unroll=2)
        def _j(j):
            bt = j // 64
            r = j % 64
            jl = r // 8
            jb = (r % 8) * 16
            rows = ytv_p[bt, jl, pl.ds(jb, 16)]
            rb = rows * _DP
            for f in range(10):
                qv_p[f, bt, jl, pl.ds(jb, 16)] = plsc.load_gather(lut, [rb + f])

        for f in range(10):
            out_h[p].append(pltpu.async_copy(
                qv_p.at[f],
                q_hbm.at[f, c, pl.ds(bt0, _BSPAN // 128)], so[p]))
    for p in (0, 1):
        for h in out_h[p]:
            h.wait()


def kernel(y, embed_table, W, b):
    yt = (y.T.astype(jnp.int32)
          .reshape(_NSTRIP, _LT, _B // 128, 128).transpose(0, 2, 1, 3))
    embp = jnp.repeat(embed_table.reshape(-1), _DP)
    wp = jnp.pad(W, ((0, 0), (0, _DP - _D))).reshape(-1)
    bp = jnp.pad(b, (0, _DP - _D))
    mesh = plsc.VectorSubcoreMesh(core_axis_name="c", subcore_axis_name="s")
    q = pl.kernel(
        _body,
        out_type=jax.ShapeDtypeStruct((_D, _NSTRIP, _B // 128, _LT, 128),
                                      jnp.float32),
        mesh=mesh,
        compiler_params=pltpu.CompilerParams(needs_layout_passes=False),
        scratch_types=[
            pltpu.VMEM((_BSPAN // 128, _LT, 128), jnp.int32),
            pltpu.VMEM((_BSPAN // 128, _LT, 128), jnp.int32),
            pltpu.VMEM((_D, _BSPAN // 128, _LT, 128), jnp.float32),
            pltpu.VMEM((_D, _BSPAN // 128, _LT, 128), jnp.float32),
            pltpu.VMEM((100 * _DP,), jnp.float32),
            pltpu.VMEM((10 * _DP,), jnp.float32),
            pltpu.VMEM((_DP,), jnp.float32),
            pltpu.VMEM((10 * _DP,), jnp.float32),
            pltpu.SemaphoreType.DMA,
            pltpu.SemaphoreType.DMA,
            pltpu.SemaphoreType.DMA,
            pltpu.SemaphoreType.DMA,
        ],
    )(yt, embp, wp, bp)
    return q.transpose(2, 4, 1, 3, 0).reshape(_B, _L, _D)


# speedup vs baseline: 1.8434x; 1.8434x over previous
import jax
import jax.numpy as jnp
from jax import lax
from jax.experimental import pallas as pl
from jax.experimental.pallas import tpu as pltpu
from jax.experimental.pallas import tpu_sc as plsc

_B, _L, _D = 16384, 200, 10
_DP = 16
_NC, _NS = 2, 16
_NW = _NC * _NS
_BSPAN = _B // _NW
_LT = 8
_NSTRIP = _L // _LT
_NCHK = _BSPAN // 16


def _build_lut(emb_hbm, w_hbm, b_hbm, embv, wv, bv, lut):
    pltpu.sync_copy(emb_hbm, embv)
    pltpu.sync_copy(w_hbm, wv)
    pltpu.sync_copy(b_hbm, bv)
    iota = lax.iota(jnp.int32, 16)
    for i in range(10):
        acc = bv[...]
        for k in range(10):
            e = embv[pl.ds((i * 10 + k) * _DP, _DP)]
            acc = acc + e * wv[pl.ds(k * _DP, _DP)]
        plsc.store_scatter(lut, [iota * _DP + i], jnp.maximum(acc, 0.0))


def _body(yt_hbm, emb_hbm, w_hbm, b_hbm, q_hbm,
          ytv0, ytv1, qv0, qv1, embv, wv, bv, lut, si0, si1, so0, so1):
    wid = lax.axis_index("s") * _NC + lax.axis_index("c")
    b0 = wid * _BSPAN
    ytv, qv, si, so = [ytv0, ytv1], [qv0, qv1], [si0, si1], [so0, so1]
    lut_f = [lut.at[pl.ds(f * _DP, _DP)] for f in range(10)]

    def start_in(c, p):
        pltpu.async_copy(
            yt_hbm.at[pl.ds(c * _LT, _LT), pl.ds(b0, _BSPAN)], ytv[p], si[p])

    def drain_in(p):
        pltpu.make_async_copy(
            yt_hbm.at[pl.ds(0, _LT), pl.ds(b0, _BSPAN)], ytv[p], si[p]).wait()

    def drain_out(p):
        pltpu.make_async_copy(
            q_hbm.at[:, pl.ds(0, _LT), pl.ds(b0, _BSPAN)], qv[p], so[p]).wait()

    def compute(p):
        ytv_p, qv_p = ytv[p], qv[p]

        @plsc.parallel_loop(0, _LT * 2)
        def _jh(jh):
            jl = jh // 2
            half = (jh % 2) * (_BSPAN // 2)
            for j in range(_NCHK // 2):
                col = half + j * 16
                rows = ytv_p[jl, pl.ds(col, 16)]
                for f in range(10):
                    qv_p[f, jl, pl.ds(col, 16)] = plsc.load_gather(
                        lut_f[f], [rows])

    def fire_out(c, p):
        for f in range(10):
            pltpu.async_copy(
                qv[p].at[f],
                q_hbm.at[f, pl.ds(c * _LT, _LT), pl.ds(b0, _BSPAN)], so[p])

    def strip(c, p, prefetch=True):
        drain_in(p)
        drain_out(p)
        compute(p)
        fire_out(c, p)
        if prefetch:
            start_in(jnp.minimum(c + 2, _NSTRIP - 1), p)

    start_in(0, 0)
    start_in(1, 1)
    fire_out(0, 0)
    fire_out(1, 1)
    _build_lut(emb_hbm, w_hbm, b_hbm, embv, wv, bv, lut)

    def pair_body(i, carry):
        strip(2 * i, 0)
        strip(2 * i + 1, 1)
        return carry

    lax.fori_loop(0, 12, pair_body, 0)
    strip(_NSTRIP - 1, 0, prefetch=False)

    drain_in(1)
    drain_out(0)
    drain_out(1)


def kernel(y, embed_table, W, b):
    yt = y.T.astype(jnp.int32)
    embp = jnp.repeat(embed_table.reshape(-1), _DP)
    wp = jnp.pad(W, ((0, 0), (0, _DP - _D))).reshape(-1)
    bp = jnp.pad(b, (0, _DP - _D))
    mesh = plsc.VectorSubcoreMesh(core_axis_name="c", subcore_axis_name="s")
    q = pl.kernel(
        _body,
        out_type=jax.ShapeDtypeStruct((_D, _L, _B), jnp.float32),
        mesh=mesh,
        compiler_params=pltpu.CompilerParams(needs_layout_passes=False),
        scratch_types=[
            pltpu.VMEM((_LT, _BSPAN), jnp.int32),
            pltpu.VMEM((_LT, _BSPAN), jnp.int32),
            pltpu.VMEM((_D, _LT, _BSPAN), jnp.float32),
            pltpu.VMEM((_D, _LT, _BSPAN), jnp.float32),
            pltpu.VMEM((100 * _DP,), jnp.float32),
            pltpu.VMEM((10 * _DP,), jnp.float32),
            pltpu.VMEM((_DP,), jnp.float32),
            pltpu.VMEM((16 * _DP,), jnp.float32),
            pltpu.SemaphoreType.DMA,
            pltpu.SemaphoreType.DMA,
            pltpu.SemaphoreType.DMA,
            pltpu.SemaphoreType.DMA,
        ],
    )(yt, embp, wp, bp)
    return q.transpose(2, 1, 0)


# speedup vs baseline: 1.9055x; 1.0337x over previous
import jax
import jax.numpy as jnp
from jax import lax
from jax.experimental import pallas as pl
from jax.experimental.pallas import tpu as pltpu
from jax.experimental.pallas import tpu_sc as plsc

_B, _L, _D = 16384, 200, 10
_DP = 16
_NC, _NS = 2, 16
_NW = _NC * _NS
_BSPAN = _B // _NW
_LT = 8
_NSTRIP = _L // _LT
_NCHK = _BSPAN // 16


def _build_lut(emb_hbm, w_hbm, b_hbm, embv, wv, bv, lut):
    pltpu.sync_copy(emb_hbm, embv)
    pltpu.sync_copy(w_hbm, wv)
    pltpu.sync_copy(b_hbm, bv)
    iota = lax.iota(jnp.int32, 16)
    for i in range(10):
        acc = bv[...]
        for k in range(10):
            e = embv[pl.ds((i * 10 + k) * _DP, _DP)]
            acc = acc + e * wv[pl.ds(k * _DP, _DP)]
        plsc.store_scatter(lut, [iota * _DP + i], jnp.maximum(acc, 0.0))


def _body(yt_hbm, emb_hbm, w_hbm, b_hbm, q_hbm,
          ytv0, ytv1, qv0, qv1, embv, wv, bv, lut, si0, si1, so0, so1):
    wid = lax.axis_index("s") * _NC + lax.axis_index("c")
    b0 = wid * _BSPAN
    ytv, qv, si, so = [ytv0, ytv1], [qv0, qv1], [si0, si1], [so0, so1]
    lut_f = [lut.at[pl.ds(f * _DP, _DP)] for f in range(10)]

    def start_in(c, p):
        pltpu.async_copy(
            yt_hbm.at[pl.ds(c * _LT, _LT), pl.ds(b0, _BSPAN)], ytv[p], si[p])

    def drain_in(p):
        pltpu.make_async_copy(
            yt_hbm.at[pl.ds(0, _LT), pl.ds(b0, _BSPAN)], ytv[p], si[p]).wait()

    def drain_out(p):
        pltpu.make_async_copy(
            q_hbm.at[:, pl.ds(0, _LT), pl.ds(b0, _BSPAN)], qv[p], so[p]).wait()

    def compute(p):
        ytv_p, qv_p = ytv[p], qv[p]

        @plsc.parallel_loop(0, _LT * 2)
        def _jh(jh):
            jl = jh // 2
            half = (jh % 2) * (_BSPAN // 2)
            for j in range(_NCHK // 2):
                col = half + j * 16
                rows = ytv_p[jl, pl.ds(col, 16)]
                for f in range(10):
                    qv_p[f, jl, pl.ds(col, 16)] = plsc.load_gather(
                        lut_f[f], [rows])

    def fire_out(c, p):
        pltpu.async_copy(
            qv[p], q_hbm.at[:, pl.ds(c * _LT, _LT), pl.ds(b0, _BSPAN)], so[p])

    def strip(c, p, prefetch=True):
        drain_in(p)
        drain_out(p)
        compute(p)
        fire_out(c, p)
        if prefetch:
            start_in(jnp.minimum(c + 2, _NSTRIP - 1), p)

    start_in(0, 0)
    start_in(1, 1)
    fire_out(0, 0)
    fire_out(1, 1)
    _build_lut(emb_hbm, w_hbm, b_hbm, embv, wv, bv, lut)

    def pair_body(i, carry):
        strip(2 * i, 0)
        strip(2 * i + 1, 1)
        return carry

    lax.fori_loop(0, 12, pair_body, 0)
    strip(_NSTRIP - 1, 0, prefetch=False)

    drain_in(1)
    drain_out(0)
    drain_out(1)


def kernel(y, embed_table, W, b):
    yt = y.T.astype(jnp.int32)
    embp = jnp.repeat(embed_table.reshape(-1), _DP)
    wp = jnp.pad(W, ((0, 0), (0, _DP - _D))).reshape(-1)
    bp = jnp.pad(b, (0, _DP - _D))
    mesh = plsc.VectorSubcoreMesh(core_axis_name="c", subcore_axis_name="s")
    q = pl.kernel(
        _body,
        out_type=jax.ShapeDtypeStruct((_D, _L, _B), jnp.float32),
        mesh=mesh,
        compiler_params=pltpu.CompilerParams(needs_layout_passes=False),
        scratch_types=[
            pltpu.VMEM((_LT, _BSPAN), jnp.int32),
            pltpu.VMEM((_LT, _BSPAN), jnp.int32),
            pltpu.VMEM((_D, _LT, _BSPAN), jnp.float32),
            pltpu.VMEM((_D, _LT, _BSPAN), jnp.float32),
            pltpu.VMEM((100 * _DP,), jnp.float32),
            pltpu.VMEM((10 * _DP,), jnp.float32),
            pltpu.VMEM((_DP,), jnp.float32),
            pltpu.VMEM((16 * _DP,), jnp.float32),
            pltpu.SemaphoreType.DMA,
            pltpu.SemaphoreType.DMA,
            pltpu.SemaphoreType.DMA,
            pltpu.SemaphoreType.DMA,
        ],
    )(yt, embp, wp, bp)
    return q.transpose(2, 1, 0)
